# baseline (speedup 1.0000x reference)
import functools

import jax
import jax.numpy as jnp
from jax import lax
from jax.experimental import pallas as pl
from jax.experimental.pallas import tpu as pltpu
from jax.experimental.pallas import tpu_sc as plsc

N = 10000
NPAD = 10240
E = 320000
F = 128
C = 64
NC = 2
NS = 16
NW = NC * NS
EPWP = 10240
EPAD = NW * EPWP
EW = 64
NWIN = EPWP // EW
NCH = 4
CH = NWIN // NCH
RPT = NPAD // NS

BN = 5120
G = NPAD // BN


@functools.lru_cache(maxsize=None)
def _sc_calls():
  mesh = plsc.VectorSubcoreMesh(
      core_axis_name="c", subcore_axis_name="s", num_cores=NC, num_subcores=NS
  )

  @functools.partial(
      pl.kernel,
      out_type=jax.ShapeDtypeStruct((NC, NPAD), jnp.float32),
      mesh=mesh,
      scratch_types=[
          pltpu.VMEM((NCH, CH, EW), jnp.int32),
          pltpu.VMEM((EW,), jnp.float32),
          pltpu.VMEM((RPT,), jnp.float32),
          pltpu.VMEM_SHARED((NPAD,), jnp.float32),
      ],
  )
  def _deg(dst_hbm, out_hbm, dst_v, ones_v, zer_v, acc):
    c = lax.axis_index("c")
    s = lax.axis_index("s")
    wid = c * NS + s

    z16 = jnp.zeros((16,), jnp.float32)
    def zb(i, carry):
      zer_v[pl.ds(i * 16, 16)] = z16
      return carry
    lax.fori_loop(0, RPT // 16, zb, 0)
    o16 = jnp.ones((16,), jnp.float32)
    def ob(i, carry):
      ones_v[pl.ds(i * 16, 16)] = o16
      return carry
    lax.fori_loop(0, EW // 16, ob, 0)
    pltpu.sync_copy(zer_v, acc.at[pl.ds(s * RPT, RPT)])
    pltpu.sync_copy(dst_hbm.at[wid], dst_v)
    plsc.subcore_barrier()

    def body(ch, carry):
      def inner(j, carry2):
        pltpu.sync_copy(ones_v, acc.at[dst_v.at[ch, j]], add=True)
        return carry2
      return lax.fori_loop(0, CH, inner, carry)
    lax.fori_loop(0, NCH, body, 0)

    plsc.subcore_barrier()
    pltpu.sync_copy(
        acc.at[pl.ds(s * RPT, RPT)], out_hbm.at[c, pl.ds(s * RPT, RPT)]
    )

  @functools.partial(
      pl.kernel,
      out_type=jax.ShapeDtypeStruct((NC, NPAD, F), jnp.float32),
      mesh=mesh,
      scratch_types=[
          pltpu.VMEM((CH, EW), jnp.int32),
          pltpu.VMEM((CH, EW), jnp.int32),
          [pltpu.VMEM((EW, F), jnp.float32)] * 4,
          pltpu.VMEM_SHARED((NPAD, F), jnp.float32),
          [pltpu.SemaphoreType.DMA] * 4,
          [pltpu.SemaphoreType.DMA] * 4,
      ],
  )
  def _seg(u_hbm, src_hbm, dst_hbm, out_hbm, src_c, dst_c, rows,
           acc, gsem, ssem):
    c = lax.axis_index("c")
    s = lax.axis_index("s")
    wid = c * NS + s

    def gather(w, buf):
      pltpu.async_copy(u_hbm.at[src_c.at[w]], rows[buf], gsem[buf])

    def gather_wait(w, buf):
      pltpu.make_async_copy(u_hbm.at[src_c.at[w]], rows[buf],
                            gsem[buf]).wait()

    def scatter(w, buf):
      pltpu.async_copy(rows[buf], acc.at[dst_c.at[w]], ssem[buf], add=True)

    def scatter_wait(w, buf):
      pltpu.make_async_copy(rows[buf], acc.at[dst_c.at[w]],
                            ssem[buf]).wait()

    def load_chunk(ch):
      pltpu.sync_copy(src_hbm.at[wid, ch], src_c)
      pltpu.sync_copy(dst_hbm.at[wid, ch], dst_c)

    def run_windows(carry):
      def body(q, carry2):
        w = 4 * q
        gather_wait(w, 0)
        scatter(w, 0)
        @pl.when(q >= 1)
        def _():
          scatter_wait(w - 2, 2)
        gather(w + 2, 2)

        gather_wait(w + 1, 1)
        scatter(w + 1, 1)
        @pl.when(q >= 1)
        def _():
          scatter_wait(w - 1, 3)
        gather(w + 3, 3)

        gather_wait(w + 2, 2)
        scatter(w + 2, 2)
        @pl.when(q < CH // 4 - 1)
        def _():
          scatter_wait(w, 0)
          gather(w + 4, 0)

        gather_wait(w + 3, 3)
        scatter(w + 3, 3)
        @pl.when(q < CH // 4 - 1)
        def _():
          scatter_wait(w + 1, 1)
          gather(w + 5, 1)
        return carry2
      lax.fori_loop(0, CH // 4, body, carry)
      scatter_wait(CH - 4, 0)
      scatter_wait(CH - 3, 1)
      scatter_wait(CH - 2, 2)
      scatter_wait(CH - 1, 3)

    z16 = jnp.zeros((16,), jnp.float32)
    def zb(k, carry):
      i = k // (F // 16)
      l = k % (F // 16)
      rows[3][i, pl.ds(l * 16, 16)] = z16
      return carry
    lax.fori_loop(0, EW * (F // 16), zb, 0)
    def zc(k, carry):
      pltpu.async_copy(rows[3], acc.at[pl.ds(s * RPT + k * EW, EW)],
                       ssem[3])
      return carry
    lax.fori_loop(0, RPT // EW, zc, 0)
    load_chunk(0)
    gather(0, 0)
    gather(1, 1)
    def zw(k, carry):
      pltpu.make_async_copy(rows[3], acc.at[pl.ds(s * RPT + k * EW, EW)],
                            ssem[3]).wait()
      return carry
    lax.fori_loop(0, RPT // EW, zw, 0)
    plsc.subcore_barrier()
    run_windows(0)

    def chunk(ch, carry):
      load_chunk(ch)
      gather(0, 0)
      gather(1, 1)
      run_windows(carry)
      return carry
    lax.fori_loop(1, NCH, chunk, 0)

    plsc.subcore_barrier()
    pltpu.sync_copy(
        acc.at[pl.ds(s * RPT, RPT)], out_hbm.at[c, pl.ds(s * RPT, RPT)]
    )

  return _deg, _seg


def _dis_body(dp_ref, dis_ref):
  deg = dp_ref[0] + dp_ref[1]
  safe = jnp.where(deg > 0, deg, 1.0)
  dis_ref[...] = jnp.where(deg > 0, lax.rsqrt(safe), 0.0)


def _u1_body(dis_ref, x_ref, w_ref, u_ref, c0_ref):
  u_ref[...] = dis_ref[...] * x_ref[...]
  c0_ref[...] = jnp.dot(x_ref[...], w_ref[0],
                        preferred_element_type=jnp.float32)


def _c2_body(h_ref, w_ref, c2_ref):
  c2_ref[...] = jnp.dot(h_ref[...], w_ref[0],
                        preferred_element_type=jnp.float32)


def _d1_body(sp_ref, dis_ref, c0_ref, w_ref, u2_ref, out1_ref):
  dis = dis_ref[...]
  tx1 = -dis * (sp_ref[0] + sp_ref[1])
  u2_ref[...] = dis * tx1
  out1_ref[...] = c0_ref[...] + jnp.dot(
      tx1, w_ref[1], preferred_element_type=jnp.float32)


def _d2_body(sp_ref, dis_ref, x_ref, o1_ref, w_ref, b_ref, h_ref, u3_ref):
  dis = dis_ref[...]
  tx2 = -2.0 * dis * (sp_ref[0] + sp_ref[1]) - x_ref[...]
  h = (o1_ref[...]
       + jnp.dot(tx2, w_ref[2], preferred_element_type=jnp.float32)
       + b_ref[...])
  h = jnp.maximum(h, 0.0)
  h_ref[...] = h
  u3_ref[...] = dis * h


def _d3_body(sp_ref, dis_ref, c2_ref, w_ref, u4_ref, out2_ref):
  dis = dis_ref[...]
  tx1 = -dis * (sp_ref[0] + sp_ref[1])
  u4_ref[...] = dis * tx1
  out2_ref[...] = c2_ref[...] + jnp.dot(
      tx1, w_ref[1], preferred_element_type=jnp.float32)


def _d4_body(sp_ref, dis_ref, h_ref, o2_ref, w_ref, b_ref, y_ref):
  dis = dis_ref[...]
  tx2 = -2.0 * dis * (sp_ref[0] + sp_ref[1]) - h_ref[...]
  logits = (o2_ref[...]
            + jnp.dot(tx2, w_ref[2], preferred_element_type=jnp.float32)
            + b_ref[...])
  m = jnp.max(logits, axis=1, keepdims=True)
  shifted = logits - m
  lse = jnp.log(jnp.sum(jnp.exp(shifted), axis=1, keepdims=True))
  y_ref[...] = shifted - lse


def _row_spec(width=F):
  return pl.BlockSpec((BN, width), lambda i: (i, 0))


def _sp_spec():
  return pl.BlockSpec((NC, BN, F), lambda i: (0, i, 0))


@functools.lru_cache(maxsize=None)
def _tc_calls():
  f32 = jnp.float32
  dis_call = pl.pallas_call(
      _dis_body,
      out_shape=jax.ShapeDtypeStruct((NPAD // F, F), f32),
  )
  w1_spec = pl.BlockSpec((3, F, F), lambda i: (0, 0, 0))
  w2_spec = pl.BlockSpec((3, F, C), lambda i: (0, 0, 0))
  u1_call = pl.pallas_call(
      _u1_body,
      grid=(G,),
      in_specs=[_row_spec(), _row_spec(), w1_spec],
      out_specs=(_row_spec(), _row_spec()),
      out_shape=(
          jax.ShapeDtypeStruct((NPAD, F), f32),
          jax.ShapeDtypeStruct((NPAD, F), f32),
      ),
  )
  c2_call = pl.pallas_call(
      _c2_body,
      grid=(G,),
      in_specs=[_row_spec(), w2_spec],
      out_specs=_row_spec(C),
      out_shape=jax.ShapeDtypeStruct((NPAD, C), f32),
  )
  d1_call = pl.pallas_call(
      _d1_body,
      grid=(G,),
      in_specs=[_sp_spec(), _row_spec(), _row_spec(), w1_spec],
      out_specs=(_row_spec(), _row_spec()),
      out_shape=(
          jax.ShapeDtypeStruct((NPAD, F), f32),
          jax.ShapeDtypeStruct((NPAD, F), f32),
      ),
  )
  d3_call = pl.pallas_call(
      _d3_body,
      grid=(G,),
      in_specs=[_sp_spec(), _row_spec(), _row_spec(C), w2_spec],
      out_specs=(_row_spec(), _row_spec(C)),
      out_shape=(
          jax.ShapeDtypeStruct((NPAD, F), f32),
          jax.ShapeDtypeStruct((NPAD, C), f32),
      ),
  )
  b1_spec = pl.BlockSpec((1, F), lambda i: (0, 0))
  d2_call = pl.pallas_call(
      _d2_body,
      grid=(G,),
      in_specs=[_sp_spec(), _row_spec(), _row_spec(), _row_spec(), w1_spec,
                b1_spec],
      out_specs=(_row_spec(), _row_spec()),
      out_shape=(
          jax.ShapeDtypeStruct((NPAD, F), f32),
          jax.ShapeDtypeStruct((NPAD, F), f32),
      ),
  )
  b2_spec = pl.BlockSpec((1, C), lambda i: (0, 0))
  d4_call = pl.pallas_call(
      _d4_body,
      grid=(G,),
      in_specs=[_sp_spec(), _row_spec(), _row_spec(), _row_spec(C), w2_spec,
                b2_spec],
      out_specs=_row_spec(C),
      out_shape=jax.ShapeDtypeStruct((NPAD, C), f32),
  )
  return (dis_call, u1_call, c2_call, d1_call, d3_call, d2_call, d4_call)


def kernel(x, edge_index, W1, b1, W2, b2):
  deg_call, seg_call = _sc_calls()
  (dis_call, u1_call, c2_call, d1_call, d3_call, d2_call,
   d4_call) = _tc_calls()

  x_pad = jnp.pad(x, ((0, NPAD - N), (0, 0)))
  pad_idx = (N + jnp.arange(EPAD - E, dtype=jnp.int32) % (NPAD - N))
  src3 = jnp.concatenate([edge_index[0], pad_idx]).reshape(NW, NCH, CH, EW)
  dst3 = jnp.concatenate([edge_index[1], pad_idx]).reshape(NW, NCH, CH, EW)

  dp = deg_call(dst3)
  dis2d = dis_call(dp.reshape(NC, NPAD // F, F))
  dis_b = jnp.broadcast_to(dis2d.reshape(NPAD, 1), (NPAD, F))

  u1, c0 = u1_call(dis_b, x_pad, W1)
  sp1 = seg_call(u1, src3, dst3)
  u2, out1 = d1_call(sp1, dis_b, c0, W1)
  sp2 = seg_call(u2, src3, dst3)
  h, u3 = d2_call(sp2, dis_b, x_pad, out1, W1, b1.reshape(1, F))
  sp3 = seg_call(u3, src3, dst3)
  c2 = c2_call(h, W2)
  u4, out2 = d3_call(sp3, dis_b, c2, W2)
  sp4 = seg_call(u4, src3, dst3)
  y = d4_call(sp4, dis_b, h, out2, W2, b2.reshape(1, C))
  return y[:N]


# speedup vs baseline: 1.0031x; 1.0031x over previous
import functools

import jax
import jax.numpy as jnp
from jax import lax
from jax.experimental import pallas as pl
from jax.experimental.pallas import tpu as pltpu
from jax.experimental.pallas import tpu_sc as plsc

N = 10000
NPAD = 10240
E = 320000
F = 128
C = 64
NC = 2
NS = 16
NW = NC * NS
EPWP = 10240
EPAD = NW * EPWP
EW = 64
NWIN = EPWP // EW
NCH = 4
CH = NWIN // NCH
RPT = NPAD // NS

BN = 5120
G = NPAD // BN


@functools.lru_cache(maxsize=None)
def _sc_calls():
  mesh = plsc.VectorSubcoreMesh(
      core_axis_name="c", subcore_axis_name="s", num_cores=NC, num_subcores=NS
  )

  @functools.partial(
      pl.kernel,
      out_type=jax.ShapeDtypeStruct((NC, NPAD), jnp.float32),
      mesh=mesh,
      scratch_types=[
          pltpu.VMEM((NCH, CH, EW), jnp.int32),
          pltpu.VMEM((EW,), jnp.float32),
          pltpu.VMEM((RPT,), jnp.float32),
          pltpu.VMEM_SHARED((NPAD,), jnp.float32),
      ],
  )
  def _deg(dst_hbm, out_hbm, dst_v, ones_v, zer_v, acc):
    c = lax.axis_index("c")
    s = lax.axis_index("s")
    wid = c * NS + s

    z16 = jnp.zeros((16,), jnp.float32)
    def zb(i, carry):
      zer_v[pl.ds(i * 16, 16)] = z16
      return carry
    lax.fori_loop(0, RPT // 16, zb, 0)
    o16 = jnp.ones((16,), jnp.float32)
    def ob(i, carry):
      ones_v[pl.ds(i * 16, 16)] = o16
      return carry
    lax.fori_loop(0, EW // 16, ob, 0)
    pltpu.sync_copy(zer_v, acc.at[pl.ds(s * RPT, RPT)])
    pltpu.sync_copy(dst_hbm.at[wid], dst_v)
    plsc.subcore_barrier()

    def body(ch, carry):
      def inner(j, carry2):
        pltpu.sync_copy(ones_v, acc.at[dst_v.at[ch, j]], add=True)
        return carry2
      return lax.fori_loop(0, CH, inner, carry)
    lax.fori_loop(0, NCH, body, 0)

    plsc.subcore_barrier()
    pltpu.sync_copy(
        acc.at[pl.ds(s * RPT, RPT)], out_hbm.at[c, pl.ds(s * RPT, RPT)]
    )

  @functools.partial(
      pl.kernel,
      out_type=jax.ShapeDtypeStruct((NC, NPAD, F), jnp.float32),
      mesh=mesh,
      scratch_types=[
          pltpu.VMEM((CH, EW), jnp.int32),
          pltpu.VMEM((CH, EW), jnp.int32),
          [pltpu.VMEM((EW, F), jnp.float32)] * 4,
          pltpu.VMEM_SHARED((NPAD, F), jnp.float32),
          [pltpu.SemaphoreType.DMA] * 4,
          [pltpu.SemaphoreType.DMA] * 4,
      ],
  )
  def _seg(u_hbm, src_hbm, dst_hbm, out_hbm, src_c, dst_c, rows,
           acc, gsem, ssem):
    c = lax.axis_index("c")
    s = lax.axis_index("s")
    wid = c * NS + s

    def gather(w, buf):
      pltpu.async_copy(u_hbm.at[src_c.at[w]], rows[buf], gsem[buf])

    def gather_wait(w, buf):
      pltpu.make_async_copy(u_hbm.at[src_c.at[w]], rows[buf],
                            gsem[buf]).wait()

    def scatter(w, buf):
      pltpu.async_copy(rows[buf], acc.at[dst_c.at[w]], ssem[buf], add=True)

    def scatter_wait(w, buf):
      pltpu.make_async_copy(rows[buf], acc.at[dst_c.at[w]],
                            ssem[buf]).wait()

    def load_chunk(ch):
      pltpu.sync_copy(src_hbm.at[wid, ch], src_c)
      pltpu.sync_copy(dst_hbm.at[wid, ch], dst_c)

    def run_windows(carry):
      def body(q, carry2):
        w = 4 * q
        gather_wait(w, 0)
        scatter(w, 0)
        @pl.when(q >= 1)
        def _():
          scatter_wait(w - 2, 2)
        gather(w + 2, 2)

        gather_wait(w + 1, 1)
        scatter(w + 1, 1)
        @pl.when(q >= 1)
        def _():
          scatter_wait(w - 1, 3)
        gather(w + 3, 3)

        gather_wait(w + 2, 2)
        scatter(w + 2, 2)
        @pl.when(q < CH // 4 - 1)
        def _():
          scatter_wait(w, 0)
          gather(w + 4, 0)

        gather_wait(w + 3, 3)
        scatter(w + 3, 3)
        @pl.when(q < CH // 4 - 1)
        def _():
          scatter_wait(w + 1, 1)
          gather(w + 5, 1)
        return carry2
      lax.fori_loop(0, CH // 4, body, carry)
      scatter_wait(CH - 4, 0)
      scatter_wait(CH - 3, 1)
      scatter_wait(CH - 2, 2)
      scatter_wait(CH - 1, 3)

    z16 = jnp.zeros((16,), jnp.float32)
    def zb(k, carry):
      i = k // (F // 16)
      l = k % (F // 16)
      rows[3][i, pl.ds(l * 16, 16)] = z16
      return carry
    lax.fori_loop(0, EW * (F // 16), zb, 0)
    def zc(k, carry):
      pltpu.async_copy(rows[3], acc.at[pl.ds(s * RPT + k * EW, EW)],
                       ssem[3])
      return carry
    lax.fori_loop(0, RPT // EW, zc, 0)
    load_chunk(0)
    gather(0, 0)
    gather(1, 1)
    def zw(k, carry):
      pltpu.make_async_copy(rows[3], acc.at[pl.ds(s * RPT + k * EW, EW)],
                            ssem[3]).wait()
      return carry
    lax.fori_loop(0, RPT // EW, zw, 0)
    plsc.subcore_barrier()
    run_windows(0)

    def chunk(ch, carry):
      load_chunk(ch)
      gather(0, 0)
      gather(1, 1)
      run_windows(carry)
      return carry
    lax.fori_loop(1, NCH, chunk, 0)

    plsc.subcore_barrier()
    pltpu.sync_copy(
        acc.at[pl.ds(s * RPT, RPT)], out_hbm.at[c, pl.ds(s * RPT, RPT)]
    )

  return _deg, _seg


def _dis_body(dp_ref, dis_ref):
  deg = dp_ref[0] + dp_ref[1]
  safe = jnp.where(deg > 0, deg, 1.0)
  dis_ref[...] = jnp.where(deg > 0, lax.rsqrt(safe), 0.0)


def _u1_body(dis_ref, x_ref, u_ref):
  u_ref[...] = dis_ref[...] * x_ref[...]


def _c0_body(x_ref, w_ref, c0_ref):
  c0_ref[...] = jnp.dot(x_ref[...], w_ref[0],
                        preferred_element_type=jnp.float32)


def _c2_body(h_ref, w_ref, c2_ref):
  c2_ref[...] = jnp.dot(h_ref[...], w_ref[0],
                        preferred_element_type=jnp.float32)


def _d1a_body(sp_ref, dis_ref, tx1_ref, u2_ref):
  dis = dis_ref[...]
  tx1 = -dis * (sp_ref[0] + sp_ref[1])
  tx1_ref[...] = tx1
  u2_ref[...] = dis * tx1


def _d1b_body(c0_ref, tx1_ref, w_ref, out1_ref):
  out1_ref[...] = c0_ref[...] + jnp.dot(
      tx1_ref[...], w_ref[1], preferred_element_type=jnp.float32)


def _d2_body(sp_ref, dis_ref, x_ref, o1_ref, w_ref, b_ref, h_ref, u3_ref):
  dis = dis_ref[...]
  tx2 = -2.0 * dis * (sp_ref[0] + sp_ref[1]) - x_ref[...]
  h = (o1_ref[...]
       + jnp.dot(tx2, w_ref[2], preferred_element_type=jnp.float32)
       + b_ref[...])
  h = jnp.maximum(h, 0.0)
  h_ref[...] = h
  u3_ref[...] = dis * h


def _d3b_body(c2_ref, tx1_ref, w_ref, out2_ref):
  out2_ref[...] = c2_ref[...] + jnp.dot(
      tx1_ref[...], w_ref[1], preferred_element_type=jnp.float32)


def _d4_body(sp_ref, dis_ref, h_ref, o2_ref, w_ref, b_ref, y_ref):
  dis = dis_ref[...]
  tx2 = -2.0 * dis * (sp_ref[0] + sp_ref[1]) - h_ref[...]
  logits = (o2_ref[...]
            + jnp.dot(tx2, w_ref[2], preferred_element_type=jnp.float32)
            + b_ref[...])
  m = jnp.max(logits, axis=1, keepdims=True)
  shifted = logits - m
  lse = jnp.log(jnp.sum(jnp.exp(shifted), axis=1, keepdims=True))
  y_ref[...] = shifted - lse


def _row_spec(width=F):
  return pl.BlockSpec((BN, width), lambda i: (i, 0))


def _sp_spec():
  return pl.BlockSpec((NC, BN, F), lambda i: (0, i, 0))


@functools.lru_cache(maxsize=None)
def _tc_calls():
  f32 = jnp.float32
  dis_call = pl.pallas_call(
      _dis_body,
      out_shape=jax.ShapeDtypeStruct((NPAD // F, F), f32),
  )
  w1_spec = pl.BlockSpec((3, F, F), lambda i: (0, 0, 0))
  w2_spec = pl.BlockSpec((3, F, C), lambda i: (0, 0, 0))
  u1_call = pl.pallas_call(
      _u1_body,
      grid=(G,),
      in_specs=[_row_spec(), _row_spec()],
      out_specs=_row_spec(),
      out_shape=jax.ShapeDtypeStruct((NPAD, F), f32),
  )
  c0_call = pl.pallas_call(
      _c0_body,
      grid=(G,),
      in_specs=[_row_spec(), w1_spec],
      out_specs=_row_spec(),
      out_shape=jax.ShapeDtypeStruct((NPAD, F), f32),
  )
  c2_call = pl.pallas_call(
      _c2_body,
      grid=(G,),
      in_specs=[_row_spec(), w2_spec],
      out_specs=_row_spec(C),
      out_shape=jax.ShapeDtypeStruct((NPAD, C), f32),
  )
  da_call = pl.pallas_call(
      _d1a_body,
      grid=(G,),
      in_specs=[_sp_spec(), _row_spec()],
      out_specs=(_row_spec(), _row_spec()),
      out_shape=(
          jax.ShapeDtypeStruct((NPAD, F), f32),
          jax.ShapeDtypeStruct((NPAD, F), f32),
      ),
  )
  d1b_call = pl.pallas_call(
      _d1b_body,
      grid=(G,),
      in_specs=[_row_spec(), _row_spec(), w1_spec],
      out_specs=_row_spec(),
      out_shape=jax.ShapeDtypeStruct((NPAD, F), f32),
  )
  d3b_call = pl.pallas_call(
      _d3b_body,
      grid=(G,),
      in_specs=[_row_spec(C), _row_spec(), w2_spec],
      out_specs=_row_spec(C),
      out_shape=jax.ShapeDtypeStruct((NPAD, C), f32),
  )
  b1_spec = pl.BlockSpec((1, F), lambda i: (0, 0))
  d2_call = pl.pallas_call(
      _d2_body,
      grid=(G,),
      in_specs=[_sp_spec(), _row_spec(), _row_spec(), _row_spec(), w1_spec,
                b1_spec],
      out_specs=(_row_spec(), _row_spec()),
      out_shape=(
          jax.ShapeDtypeStruct((NPAD, F), f32),
          jax.ShapeDtypeStruct((NPAD, F), f32),
      ),
  )
  b2_spec = pl.BlockSpec((1, C), lambda i: (0, 0))
  d4_call = pl.pallas_call(
      _d4_body,
      grid=(G,),
      in_specs=[_sp_spec(), _row_spec(), _row_spec(), _row_spec(C), w2_spec,
                b2_spec],
      out_specs=_row_spec(C),
      out_shape=jax.ShapeDtypeStruct((NPAD, C), f32),
  )
  return (dis_call, u1_call, c0_call, c2_call, da_call, d1b_call,
          d3b_call, d2_call, d4_call)


def kernel(x, edge_index, W1, b1, W2, b2):
  deg_call, seg_call = _sc_calls()
  (dis_call, u1_call, c0_call, c2_call, da_call, d1b_call, d3b_call,
   d2_call, d4_call) = _tc_calls()

  x_pad = jnp.pad(x, ((0, NPAD - N), (0, 0)))
  pad_idx = (N + jnp.arange(EPAD - E, dtype=jnp.int32) % (NPAD - N))
  src3 = jnp.concatenate([edge_index[0], pad_idx]).reshape(NW, NCH, CH, EW)
  dst3 = jnp.concatenate([edge_index[1], pad_idx]).reshape(NW, NCH, CH, EW)

  dp = deg_call(dst3)
  dis2d = dis_call(dp.reshape(NC, NPAD // F, F))
  dis_b = jnp.broadcast_to(dis2d.reshape(NPAD, 1), (NPAD, F))

  u1 = u1_call(dis_b, x_pad)
  sp1 = seg_call(u1, src3, dst3)
  c0 = c0_call(x_pad, W1)
  tx1, u2 = da_call(sp1, dis_b)
  sp2 = seg_call(u2, src3, dst3)
  out1 = d1b_call(c0, tx1, W1)
  h, u3 = d2_call(sp2, dis_b, x_pad, out1, W1, b1.reshape(1, F))
  sp3 = seg_call(u3, src3, dst3)
  c2 = c2_call(h, W2)
  tx1b, u4 = da_call(sp3, dis_b)
  sp4 = seg_call(u4, src3, dst3)
  out2 = d3b_call(c2, tx1b, W2)
  y = d4_call(sp4, dis_b, h, out2, W2, b2.reshape(1, C))
  return y[:N]


# speedup vs baseline: 1.0201x; 1.0170x over previous
import functools

import jax
import jax.numpy as jnp
from jax import lax
from jax.experimental import pallas as pl
from jax.experimental.pallas import tpu as pltpu
from jax.experimental.pallas import tpu_sc as plsc

N = 10000
NPAD = 10240
E = 320000
F = 128
C = 64
NC = 2
NS = 16
NW = NC * NS
EPWP = 10240
EPAD = NW * EPWP
EW = 64
NWIN = EPWP // EW
NCH = 4
CH = NWIN // NCH
RPT = NPAD // NS

BN = 5120
G = NPAD // BN


@functools.lru_cache(maxsize=None)
def _sc_calls():
  mesh = plsc.VectorSubcoreMesh(
      core_axis_name="c", subcore_axis_name="s", num_cores=NC, num_subcores=NS
  )

  @functools.partial(
      pl.kernel,
      out_type=jax.ShapeDtypeStruct((NC, NPAD), jnp.float32),
      mesh=mesh,
      scratch_types=[
          pltpu.VMEM((NCH, CH, EW), jnp.int32),
          pltpu.VMEM((EW,), jnp.float32),
          pltpu.VMEM((RPT,), jnp.float32),
          pltpu.VMEM_SHARED((NPAD,), jnp.float32),
          pltpu.SemaphoreType.DMA,
      ],
  )
  def _deg(dst_hbm, out_hbm, dst_v, ones_v, zer_v, acc, dsem):
    c = lax.axis_index("c")
    s = lax.axis_index("s")
    wid = c * NS + s

    z16 = jnp.zeros((16,), jnp.float32)
    def zb(i, carry):
      zer_v[pl.ds(i * 16, 16)] = z16
      return carry
    lax.fori_loop(0, RPT // 16, zb, 0)
    o16 = jnp.ones((16,), jnp.float32)
    def ob(i, carry):
      ones_v[pl.ds(i * 16, 16)] = o16
      return carry
    lax.fori_loop(0, EW // 16, ob, 0)
    pltpu.sync_copy(zer_v, acc.at[pl.ds(s * RPT, RPT)])
    pltpu.sync_copy(dst_hbm.at[wid], dst_v)
    plsc.subcore_barrier()

    def body(ch, carry):
      def inner(j, carry2):
        pltpu.async_copy(ones_v, acc.at[dst_v.at[ch, j]], dsem, add=True)
        return carry2
      return lax.fori_loop(0, CH, inner, carry)
    lax.fori_loop(0, NCH, body, 0)
    def drain(k, carry):
      pltpu.make_async_copy(ones_v, acc.at[dst_v.at[0, 0]], dsem).wait()
      return carry
    lax.fori_loop(0, NWIN, drain, 0)

    plsc.subcore_barrier()
    pltpu.sync_copy(
        acc.at[pl.ds(s * RPT, RPT)], out_hbm.at[c, pl.ds(s * RPT, RPT)]
    )

  @functools.partial(
      pl.kernel,
      out_type=jax.ShapeDtypeStruct((NC, NPAD, F), jnp.float32),
      mesh=mesh,
      scratch_types=[
          pltpu.VMEM((CH, EW), jnp.int32),
          pltpu.VMEM((CH, EW), jnp.int32),
          [pltpu.VMEM((EW, F), jnp.float32)] * 4,
          pltpu.VMEM_SHARED((NPAD, F), jnp.float32),
          [pltpu.SemaphoreType.DMA] * 4,
          [pltpu.SemaphoreType.DMA] * 4,
      ],
  )
  def _seg(u_hbm, src_hbm, dst_hbm, out_hbm, src_c, dst_c, rows,
           acc, gsem, ssem):
    c = lax.axis_index("c")
    s = lax.axis_index("s")
    wid = c * NS + s

    def gather(w, buf):
      pltpu.async_copy(u_hbm.at[src_c.at[w]], rows[buf], gsem[buf])

    def gather_wait(w, buf):
      pltpu.make_async_copy(u_hbm.at[src_c.at[w]], rows[buf],
                            gsem[buf]).wait()

    def scatter(w, buf):
      pltpu.async_copy(rows[buf], acc.at[dst_c.at[w]], ssem[buf], add=True)

    def scatter_wait(w, buf):
      pltpu.make_async_copy(rows[buf], acc.at[dst_c.at[w]],
                            ssem[buf]).wait()

    def load_chunk(ch):
      pltpu.sync_copy(src_hbm.at[wid, ch], src_c)
      pltpu.sync_copy(dst_hbm.at[wid, ch], dst_c)

    def run_windows(carry):
      def body(q, carry2):
        w = 4 * q
        gather_wait(w, 0)
        scatter(w, 0)
        @pl.when(q >= 1)
        def _():
          scatter_wait(w - 2, 2)
        gather(w + 2, 2)

        gather_wait(w + 1, 1)
        scatter(w + 1, 1)
        @pl.when(q >= 1)
        def _():
          scatter_wait(w - 1, 3)
        gather(w + 3, 3)

        gather_wait(w + 2, 2)
        scatter(w + 2, 2)
        @pl.when(q < CH // 4 - 1)
        def _():
          scatter_wait(w, 0)
          gather(w + 4, 0)

        gather_wait(w + 3, 3)
        scatter(w + 3, 3)
        @pl.when(q < CH // 4 - 1)
        def _():
          scatter_wait(w + 1, 1)
          gather(w + 5, 1)
        return carry2
      lax.fori_loop(0, CH // 4, body, carry)
      scatter_wait(CH - 4, 0)
      scatter_wait(CH - 3, 1)
      scatter_wait(CH - 2, 2)
      scatter_wait(CH - 1, 3)

    z16 = jnp.zeros((16,), jnp.float32)
    def zb(k, carry):
      i = k // (F // 16)
      l = k % (F // 16)
      rows[3][i, pl.ds(l * 16, 16)] = z16
      return carry
    lax.fori_loop(0, EW * (F // 16), zb, 0)
    def zc(k, carry):
      pltpu.async_copy(rows[3], acc.at[pl.ds(s * RPT + k * EW, EW)],
                       ssem[3])
      return carry
    lax.fori_loop(0, RPT // EW, zc, 0)
    load_chunk(0)
    gather(0, 0)
    gather(1, 1)
    def zw(k, carry):
      pltpu.make_async_copy(rows[3], acc.at[pl.ds(s * RPT + k * EW, EW)],
                            ssem[3]).wait()
      return carry
    lax.fori_loop(0, RPT // EW, zw, 0)
    plsc.subcore_barrier()
    run_windows(0)

    def chunk(ch, carry):
      load_chunk(ch)
      gather(0, 0)
      gather(1, 1)
      run_windows(carry)
      return carry
    lax.fori_loop(1, NCH, chunk, 0)

    plsc.subcore_barrier()
    pltpu.sync_copy(
        acc.at[pl.ds(s * RPT, RPT)], out_hbm.at[c, pl.ds(s * RPT, RPT)]
    )

  return _deg, _seg


def _dis_body(dp_ref, dis_ref):
  deg = dp_ref[0] + dp_ref[1]
  safe = jnp.where(deg > 0, deg, 1.0)
  dis_ref[...] = jnp.where(deg > 0, lax.rsqrt(safe), 0.0)


def _u1_body(dis_ref, x_ref, u_ref):
  u_ref[...] = dis_ref[...] * x_ref[...]


def _c0_body(x_ref, w_ref, c0_ref):
  c0_ref[...] = jnp.dot(x_ref[...], w_ref[0],
                        preferred_element_type=jnp.float32)


def _c2_body(h_ref, w_ref, c2_ref):
  c2_ref[...] = jnp.dot(h_ref[...], w_ref[0],
                        preferred_element_type=jnp.float32)


def _d1a_body(sp_ref, dis_ref, tx1_ref, u2_ref):
  dis = dis_ref[...]
  tx1 = -dis * (sp_ref[0] + sp_ref[1])
  tx1_ref[...] = tx1
  u2_ref[...] = dis * tx1


def _d1b_body(c0_ref, tx1_ref, w_ref, out1_ref):
  out1_ref[...] = c0_ref[...] + jnp.dot(
      tx1_ref[...], w_ref[1], preferred_element_type=jnp.float32)


def _d2_body(sp_ref, dis_ref, x_ref, o1_ref, w_ref, b_ref, h_ref, u3_ref):
  dis = dis_ref[...]
  tx2 = -2.0 * dis * (sp_ref[0] + sp_ref[1]) - x_ref[...]
  h = (o1_ref[...]
       + jnp.dot(tx2, w_ref[2], preferred_element_type=jnp.float32)
       + b_ref[...])
  h = jnp.maximum(h, 0.0)
  h_ref[...] = h
  u3_ref[...] = dis * h


def _d3b_body(c2_ref, tx1_ref, w_ref, out2_ref):
  out2_ref[...] = c2_ref[...] + jnp.dot(
      tx1_ref[...], w_ref[1], preferred_element_type=jnp.float32)


def _d4_body(sp_ref, dis_ref, h_ref, o2_ref, w_ref, b_ref, y_ref):
  dis = dis_ref[...]
  tx2 = -2.0 * dis * (sp_ref[0] + sp_ref[1]) - h_ref[...]
  logits = (o2_ref[...]
            + jnp.dot(tx2, w_ref[2], preferred_element_type=jnp.float32)
            + b_ref[...])
  m = jnp.max(logits, axis=1, keepdims=True)
  shifted = logits - m
  lse = jnp.log(jnp.sum(jnp.exp(shifted), axis=1, keepdims=True))
  y_ref[...] = shifted - lse


def _row_spec(width=F):
  return pl.BlockSpec((BN, width), lambda i: (i, 0))


def _sp_spec():
  return pl.BlockSpec((NC, BN, F), lambda i: (0, i, 0))


@functools.lru_cache(maxsize=None)
def _tc_calls():
  f32 = jnp.float32
  dis_call = pl.pallas_call(
      _dis_body,
      out_shape=jax.ShapeDtypeStruct((NPAD // F, F), f32),
  )
  w1_spec = pl.BlockSpec((3, F, F), lambda i: (0, 0, 0))
  w2_spec = pl.BlockSpec((3, F, C), lambda i: (0, 0, 0))
  u1_call = pl.pallas_call(
      _u1_body,
      grid=(G,),
      in_specs=[_row_spec(), _row_spec()],
      out_specs=_row_spec(),
      out_shape=jax.ShapeDtypeStruct((NPAD, F), f32),
  )
  c0_call = pl.pallas_call(
      _c0_body,
      grid=(G,),
      in_specs=[_row_spec(), w1_spec],
      out_specs=_row_spec(),
      out_shape=jax.ShapeDtypeStruct((NPAD, F), f32),
  )
  c2_call = pl.pallas_call(
      _c2_body,
      grid=(G,),
      in_specs=[_row_spec(), w2_spec],
      out_specs=_row_spec(C),
      out_shape=jax.ShapeDtypeStruct((NPAD, C), f32),
  )
  da_call = pl.pallas_call(
      _d1a_body,
      grid=(G,),
      in_specs=[_sp_spec(), _row_spec()],
      out_specs=(_row_spec(), _row_spec()),
      out_shape=(
          jax.ShapeDtypeStruct((NPAD, F), f32),
          jax.ShapeDtypeStruct((NPAD, F), f32),
      ),
  )
  d1b_call = pl.pallas_call(
      _d1b_body,
      grid=(G,),
      in_specs=[_row_spec(), _row_spec(), w1_spec],
      out_specs=_row_spec(),
      out_shape=jax.ShapeDtypeStruct((NPAD, F), f32),
  )
  d3b_call = pl.pallas_call(
      _d3b_body,
      grid=(G,),
      in_specs=[_row_spec(C), _row_spec(), w2_spec],
      out_specs=_row_spec(C),
      out_shape=jax.ShapeDtypeStruct((NPAD, C), f32),
  )
  b1_spec = pl.BlockSpec((1, F), lambda i: (0, 0))
  d2_call = pl.pallas_call(
      _d2_body,
      grid=(G,),
      in_specs=[_sp_spec(), _row_spec(), _row_spec(), _row_spec(), w1_spec,
                b1_spec],
      out_specs=(_row_spec(), _row_spec()),
      out_shape=(
          jax.ShapeDtypeStruct((NPAD, F), f32),
          jax.ShapeDtypeStruct((NPAD, F), f32),
      ),
  )
  b2_spec = pl.BlockSpec((1, C), lambda i: (0, 0))
  d4_call = pl.pallas_call(
      _d4_body,
      grid=(G,),
      in_specs=[_sp_spec(), _row_spec(), _row_spec(), _row_spec(C), w2_spec,
                b2_spec],
      out_specs=_row_spec(C),
      out_shape=jax.ShapeDtypeStruct((NPAD, C), f32),
  )
  return (dis_call, u1_call, c0_call, c2_call, da_call, d1b_call,
          d3b_call, d2_call, d4_call)


def kernel(x, edge_index, W1, b1, W2, b2):
  deg_call, seg_call = _sc_calls()
  (dis_call, u1_call, c0_call, c2_call, da_call, d1b_call, d3b_call,
   d2_call, d4_call) = _tc_calls()

  x_pad = jnp.pad(x, ((0, NPAD - N), (0, 0)))
  pad_idx = (N + jnp.arange(EPAD - E, dtype=jnp.int32) % (NPAD - N))
  src3 = jnp.concatenate([edge_index[0], pad_idx]).reshape(NW, NCH, CH, EW)
  dst3 = jnp.concatenate([edge_index[1], pad_idx]).reshape(NW, NCH, CH, EW)

  dp = deg_call(dst3)
  dis2d = dis_call(dp.reshape(NC, NPAD // F, F))
  dis_b = jnp.broadcast_to(dis2d.reshape(NPAD, 1), (NPAD, F))

  u1 = u1_call(dis_b, x_pad)
  sp1 = seg_call(u1, src3, dst3)
  c0 = c0_call(x_pad, W1)
  tx1, u2 = da_call(sp1, dis_b)
  sp2 = seg_call(u2, src3, dst3)
  out1 = d1b_call(c0, tx1, W1)
  h, u3 = d2_call(sp2, dis_b, x_pad, out1, W1, b1.reshape(1, F))
  sp3 = seg_call(u3, src3, dst3)
  c2 = c2_call(h, W2)
  tx1b, u4 = da_call(sp3, dis_b)
  sp4 = seg_call(u4, src3, dst3)
  out2 = d3b_call(c2, tx1b, W2)
  y = d4_call(sp4, dis_b, h, out2, W2, b2.reshape(1, C))
  return y[:N]


# speedup vs baseline: 1.0273x; 1.0071x over previous
import functools

import jax
import jax.numpy as jnp
from jax import lax
from jax.experimental import pallas as pl
from jax.experimental.pallas import tpu as pltpu
from jax.experimental.pallas import tpu_sc as plsc

N = 10000
NPAD = 10240
E = 320000
F = 128
C = 64
NC = 2
NS = 16
NW = NC * NS
EPWP = 10240
EPAD = NW * EPWP
EW = 64
NWIN = EPWP // EW
NCH = 4
CH = NWIN // NCH
RPT = NPAD // NS

BN = 5120
G = NPAD // BN


@functools.lru_cache(maxsize=None)
def _sc_calls():
  mesh = plsc.VectorSubcoreMesh(
      core_axis_name="c", subcore_axis_name="s", num_cores=NC, num_subcores=NS
  )

  @functools.partial(
      pl.kernel,
      out_type=jax.ShapeDtypeStruct((NC, NPAD), jnp.float32),
      mesh=mesh,
      scratch_types=[
          pltpu.VMEM((NCH, CH, EW), jnp.int32),
          pltpu.VMEM((EW,), jnp.float32),
          pltpu.VMEM((RPT,), jnp.float32),
          pltpu.VMEM_SHARED((NPAD,), jnp.float32),
          pltpu.SemaphoreType.DMA,
      ],
  )
  def _deg(dst_hbm, out_hbm, dst_v, ones_v, zer_v, acc, dsem):
    c = lax.axis_index("c")
    s = lax.axis_index("s")
    wid = c * NS + s

    z16 = jnp.zeros((16,), jnp.float32)
    def zb(i, carry):
      zer_v[pl.ds(i * 16, 16)] = z16
      return carry
    lax.fori_loop(0, RPT // 16, zb, 0)
    o16 = jnp.ones((16,), jnp.float32)
    def ob(i, carry):
      ones_v[pl.ds(i * 16, 16)] = o16
      return carry
    lax.fori_loop(0, EW // 16, ob, 0)
    pltpu.sync_copy(zer_v, acc.at[pl.ds(s * RPT, RPT)])
    pltpu.sync_copy(dst_hbm.at[wid], dst_v)
    plsc.subcore_barrier()

    def body(ch, carry):
      def inner(j, carry2):
        pltpu.async_copy(ones_v, acc.at[dst_v.at[ch, j]], dsem, add=True)
        return carry2
      return lax.fori_loop(0, CH, inner, carry)
    lax.fori_loop(0, NCH, body, 0)
    def drain(k, carry):
      pltpu.make_async_copy(ones_v, acc.at[dst_v.at[0, 0]], dsem).wait()
      return carry
    lax.fori_loop(0, NWIN, drain, 0)

    plsc.subcore_barrier()
    pltpu.sync_copy(
        acc.at[pl.ds(s * RPT, RPT)], out_hbm.at[c, pl.ds(s * RPT, RPT)]
    )

  @functools.partial(
      pl.kernel,
      out_type=jax.ShapeDtypeStruct((NC, NPAD, F), jnp.float32),
      mesh=mesh,
      scratch_types=[
          [pltpu.VMEM((CH, EW), jnp.int32)] * 2,
          pltpu.VMEM((CH, EW), jnp.int32),
          [pltpu.VMEM((EW, F), jnp.float32)] * 4,
          pltpu.VMEM_SHARED((NPAD, F), jnp.float32),
          [pltpu.SemaphoreType.DMA] * 4,
          [pltpu.SemaphoreType.DMA] * 4,
          pltpu.SemaphoreType.DMA,
      ],
  )
  def _seg(u_hbm, src_hbm, dst_hbm, out_hbm, src_cs, dst_c, rows,
           acc, gsem, ssem, psem):
    c = lax.axis_index("c")
    s = lax.axis_index("s")
    wid = c * NS + s

    def gather(w, buf, sc):
      pltpu.async_copy(u_hbm.at[sc.at[w]], rows[buf], gsem[buf])

    def gather_wait(w, buf, sc):
      pltpu.make_async_copy(u_hbm.at[sc.at[w]], rows[buf],
                            gsem[buf]).wait()

    def scatter(w, buf):
      pltpu.async_copy(rows[buf], acc.at[dst_c.at[w]], ssem[buf], add=True)

    def scatter_wait(w, buf):
      pltpu.make_async_copy(rows[buf], acc.at[dst_c.at[w]],
                            ssem[buf]).wait()

    def run_windows(sc):
      def body(q, carry2):
        w = 4 * q
        gather_wait(w, 0, sc)
        scatter(w, 0)
        @pl.when(q >= 1)
        def _():
          scatter_wait(w - 2, 2)
        gather(w + 2, 2, sc)

        gather_wait(w + 1, 1, sc)
        scatter(w + 1, 1)
        @pl.when(q >= 1)
        def _():
          scatter_wait(w - 1, 3)
        gather(w + 3, 3, sc)

        gather_wait(w + 2, 2, sc)
        scatter(w + 2, 2)
        @pl.when(q < CH // 4 - 1)
        def _():
          scatter_wait(w, 0)
          gather(w + 4, 0, sc)

        gather_wait(w + 3, 3, sc)
        scatter(w + 3, 3)
        @pl.when(q < CH // 4 - 1)
        def _():
          scatter_wait(w + 1, 1)
          gather(w + 5, 1, sc)
        return carry2
      lax.fori_loop(0, CH // 4, body, 0)
      scatter_wait(CH - 4, 0)
      scatter_wait(CH - 3, 1)
      scatter_wait(CH - 2, 2)
      scatter_wait(CH - 1, 3)

    z16 = jnp.zeros((16,), jnp.float32)
    def zb(k, carry):
      i = k // (F // 16)
      l = k % (F // 16)
      rows[3][i, pl.ds(l * 16, 16)] = z16
      return carry
    lax.fori_loop(0, EW * (F // 16), zb, 0)
    def zc(k, carry):
      pltpu.async_copy(rows[3], acc.at[pl.ds(s * RPT + k * EW, EW)],
                       ssem[3])
      return carry
    lax.fori_loop(0, RPT // EW, zc, 0)
    pltpu.sync_copy(src_hbm.at[wid, 0], src_cs[0])
    pltpu.sync_copy(dst_hbm.at[wid, 0], dst_c)
    gather(0, 0, src_cs[0])
    gather(1, 1, src_cs[0])
    def zw(k, carry):
      pltpu.make_async_copy(rows[3], acc.at[pl.ds(s * RPT + k * EW, EW)],
                            ssem[3]).wait()
      return carry
    lax.fori_loop(0, RPT // EW, zw, 0)
    plsc.subcore_barrier()

    for ch in range(NCH):
      sc = src_cs[ch % 2]
      if ch > 0:
        pltpu.make_async_copy(src_hbm.at[wid, ch], sc, psem).wait()
        pltpu.sync_copy(dst_hbm.at[wid, ch], dst_c)
        gather(0, 0, sc)
        gather(1, 1, sc)
      if ch < NCH - 1:
        pltpu.async_copy(src_hbm.at[wid, ch + 1], src_cs[(ch + 1) % 2],
                         psem)
      run_windows(sc)

    plsc.subcore_barrier()
    pltpu.sync_copy(
        acc.at[pl.ds(s * RPT, RPT)], out_hbm.at[c, pl.ds(s * RPT, RPT)]
    )

  return _deg, _seg


def _dis_body(dp_ref, dis_ref):
  deg = dp_ref[0] + dp_ref[1]
  safe = jnp.where(deg > 0, deg, 1.0)
  dis_ref[...] = jnp.where(deg > 0, lax.rsqrt(safe), 0.0)


def _u1_body(dis_ref, x_ref, u_ref):
  u_ref[...] = dis_ref[...] * x_ref[...]


def _c0_body(x_ref, w_ref, c0_ref):
  c0_ref[...] = jnp.dot(x_ref[...], w_ref[0],
                        preferred_element_type=jnp.float32)


def _c2_body(h_ref, w_ref, c2_ref):
  c2_ref[...] = jnp.dot(h_ref[...], w_ref[0],
                        preferred_element_type=jnp.float32)


def _d1a_body(sp_ref, dis_ref, tx1_ref, u2_ref):
  dis = dis_ref[...]
  tx1 = -dis * (sp_ref[0] + sp_ref[1])
  tx1_ref[...] = tx1
  u2_ref[...] = dis * tx1


def _d1b_body(c0_ref, tx1_ref, w_ref, out1_ref):
  out1_ref[...] = c0_ref[...] + jnp.dot(
      tx1_ref[...], w_ref[1], preferred_element_type=jnp.float32)


def _d2_body(sp_ref, dis_ref, x_ref, o1_ref, w_ref, b_ref, h_ref, u3_ref):
  dis = dis_ref[...]
  tx2 = -2.0 * dis * (sp_ref[0] + sp_ref[1]) - x_ref[...]
  h = (o1_ref[...]
       + jnp.dot(tx2, w_ref[2], preferred_element_type=jnp.float32)
       + b_ref[...])
  h = jnp.maximum(h, 0.0)
  h_ref[...] = h
  u3_ref[...] = dis * h


def _d3b_body(c2_ref, tx1_ref, w_ref, out2_ref):
  out2_ref[...] = c2_ref[...] + jnp.dot(
      tx1_ref[...], w_ref[1], preferred_element_type=jnp.float32)


def _d4_body(sp_ref, dis_ref, h_ref, o2_ref, w_ref, b_ref, y_ref):
  dis = dis_ref[...]
  tx2 = -2.0 * dis * (sp_ref[0] + sp_ref[1]) - h_ref[...]
  logits = (o2_ref[...]
            + jnp.dot(tx2, w_ref[2], preferred_element_type=jnp.float32)
            + b_ref[...])
  m = jnp.max(logits, axis=1, keepdims=True)
  shifted = logits - m
  lse = jnp.log(jnp.sum(jnp.exp(shifted), axis=1, keepdims=True))
  y_ref[...] = shifted - lse


def _row_spec(width=F):
  return pl.BlockSpec((BN, width), lambda i: (i, 0))


def _sp_spec():
  return pl.BlockSpec((NC, BN, F), lambda i: (0, i, 0))


@functools.lru_cache(maxsize=None)
def _tc_calls():
  f32 = jnp.float32
  dis_call = pl.pallas_call(
      _dis_body,
      out_shape=jax.ShapeDtypeStruct((NPAD // F, F), f32),
  )
  w1_spec = pl.BlockSpec((3, F, F), lambda i: (0, 0, 0))
  w2_spec = pl.BlockSpec((3, F, C), lambda i: (0, 0, 0))
  u1_call = pl.pallas_call(
      _u1_body,
      grid=(G,),
      in_specs=[_row_spec(), _row_spec()],
      out_specs=_row_spec(),
      out_shape=jax.ShapeDtypeStruct((NPAD, F), f32),
  )
  c0_call = pl.pallas_call(
      _c0_body,
      grid=(G,),
      in_specs=[_row_spec(), w1_spec],
      out_specs=_row_spec(),
      out_shape=jax.ShapeDtypeStruct((NPAD, F), f32),
  )
  c2_call = pl.pallas_call(
      _c2_body,
      grid=(G,),
      in_specs=[_row_spec(), w2_spec],
      out_specs=_row_spec(C),
      out_shape=jax.ShapeDtypeStruct((NPAD, C), f32),
  )
  da_call = pl.pallas_call(
      _d1a_body,
      grid=(G,),
      in_specs=[_sp_spec(), _row_spec()],
      out_specs=(_row_spec(), _row_spec()),
      out_shape=(
          jax.ShapeDtypeStruct((NPAD, F), f32),
          jax.ShapeDtypeStruct((NPAD, F), f32),
      ),
  )
  d1b_call = pl.pallas_call(
      _d1b_body,
      grid=(G,),
      in_specs=[_row_spec(), _row_spec(), w1_spec],
      out_specs=_row_spec(),
      out_shape=jax.ShapeDtypeStruct((NPAD, F), f32),
  )
  d3b_call = pl.pallas_call(
      _d3b_body,
      grid=(G,),
      in_specs=[_row_spec(C), _row_spec(), w2_spec],
      out_specs=_row_spec(C),
      out_shape=jax.ShapeDtypeStruct((NPAD, C), f32),
  )
  b1_spec = pl.BlockSpec((1, F), lambda i: (0, 0))
  d2_call = pl.pallas_call(
      _d2_body,
      grid=(G,),
      in_specs=[_sp_spec(), _row_spec(), _row_spec(), _row_spec(), w1_spec,
                b1_spec],
      out_specs=(_row_spec(), _row_spec()),
      out_shape=(
          jax.ShapeDtypeStruct((NPAD, F), f32),
          jax.ShapeDtypeStruct((NPAD, F), f32),
      ),
  )
  b2_spec = pl.BlockSpec((1, C), lambda i: (0, 0))
  d4_call = pl.pallas_call(
      _d4_body,
      grid=(G,),
      in_specs=[_sp_spec(), _row_spec(), _row_spec(), _row_spec(C), w2_spec,
                b2_spec],
      out_specs=_row_spec(C),
      out_shape=jax.ShapeDtypeStruct((NPAD, C), f32),
  )
  return (dis_call, u1_call, c0_call, c2_call, da_call, d1b_call,
          d3b_call, d2_call, d4_call)


def kernel(x, edge_index, W1, b1, W2, b2):
  deg_call, seg_call = _sc_calls()
  (dis_call, u1_call, c0_call, c2_call, da_call, d1b_call, d3b_call,
   d2_call, d4_call) = _tc_calls()

  x_pad = jnp.pad(x, ((0, NPAD - N), (0, 0)))
  pad_idx = (N + jnp.arange(EPAD - E, dtype=jnp.int32) % (NPAD - N))
  src3 = jnp.concatenate([edge_index[0], pad_idx]).reshape(NW, NCH, CH, EW)
  dst3 = jnp.concatenate([edge_index[1], pad_idx]).reshape(NW, NCH, CH, EW)

  dp = deg_call(dst3)
  dis2d = dis_call(dp.reshape(NC, NPAD // F, F))
  dis_b = jnp.broadcast_to(dis2d.reshape(NPAD, 1), (NPAD, F))

  u1 = u1_call(dis_b, x_pad)
  sp1 = seg_call(u1, src3, dst3)
  c0 = c0_call(x_pad, W1)
  tx1, u2 = da_call(sp1, dis_b)
  sp2 = seg_call(u2, src3, dst3)
  out1 = d1b_call(c0, tx1, W1)
  h, u3 = d2_call(sp2, dis_b, x_pad, out1, W1, b1.reshape(1, F))
  sp3 = seg_call(u3, src3, dst3)
  c2 = c2_call(h, W2)
  tx1b, u4 = da_call(sp3, dis_b)
  sp4 = seg_call(u4, src3, dst3)
  out2 = d3b_call(c2, tx1b, W2)
  y = d4_call(sp4, dis_b, h, out2, W2, b2.reshape(1, C))
  return y[:N]
